# Initial kernel scaffold; baseline (speedup 1.0000x reference)
import jax
import jax.numpy as jnp
from jax.experimental import pallas as pl


def kernel(features, edge_index, W0, b0, W1, b1, W2, b2):
    raise NotImplementedError("write your pallas kernel here")


# speedup vs baseline: 5.3505x; 5.3505x over previous
import functools

import jax
import jax.numpy as jnp
from jax import lax
from jax.experimental import pallas as pl
from jax.experimental.pallas import tpu as pltpu
from jax.experimental.pallas import tpu_sc as plsc

N_NODES = 10000
N_EDGES = 320000
NC = 2
NS = 16
NW = NC * NS
CHUNK = 128
E_PAD = -(-N_EDGES // (NW * CHUNK)) * (NW * CHUNK)
EPW = E_PAD // NW
N_CHUNKS = EPW // CHUNK
AGG_ROWS = 10240
RPT = AGG_ROWS // NS
RB = RPT // CHUNK


def _mesh():
    return plsc.VectorSubcoreMesh(core_axis_name="c", subcore_axis_name="s")


@functools.partial(
    pl.kernel,
    mesh=_mesh(),
    out_type=jax.ShapeDtypeStruct((NC * 2 * AGG_ROWS,), jnp.float32),
    scratch_types=[
        pltpu.VMEM((CHUNK,), jnp.int32),
        pltpu.VMEM((CHUNK,), jnp.float32),
        pltpu.VMEM((CHUNK,), jnp.float32),
        pltpu.VMEM_SHARED((AGG_ROWS,), jnp.float32),
        pltpu.VMEM_SHARED((AGG_ROWS,), jnp.float32),
    ],
)
def _deg_kernel(src_hbm, dst_hbm, out_hbm, idx_v, ones_v, zbuf_v, hs_sh, hd_sh):
    cid = lax.axis_index("c")
    sid = lax.axis_index("s")
    wid = sid * NC + cid
    r0 = sid * RPT
    for j in range(CHUNK // 16):
        ones_v[pl.ds(j * 16, 16)] = jnp.full((16,), 1.0, jnp.float32)
        zbuf_v[pl.ds(j * 16, 16)] = jnp.zeros((16,), jnp.float32)
    for j in range(RB):
        pltpu.sync_copy(zbuf_v, hs_sh.at[pl.ds(r0 + j * CHUNK, CHUNK)])
        pltpu.sync_copy(zbuf_v, hd_sh.at[pl.ds(r0 + j * CHUNK, CHUNK)])
    plsc.subcore_barrier()

    def body(i, carry):
        off = wid * EPW + i * CHUNK
        pltpu.sync_copy(src_hbm.at[pl.ds(off, CHUNK)], idx_v)
        pltpu.sync_copy(ones_v, hs_sh.at[idx_v], add=True)
        pltpu.sync_copy(dst_hbm.at[pl.ds(off, CHUNK)], idx_v)
        pltpu.sync_copy(ones_v, hd_sh.at[idx_v], add=True)
        return carry

    lax.fori_loop(0, N_CHUNKS, body, 0)
    plsc.subcore_barrier()
    obase = cid * (2 * AGG_ROWS)
    for j in range(RB):
        pltpu.sync_copy(hs_sh.at[pl.ds(r0 + j * CHUNK, CHUNK)], zbuf_v)
        pltpu.sync_copy(zbuf_v, out_hbm.at[pl.ds(obase + r0 + j * CHUNK, CHUNK)])
        pltpu.sync_copy(hd_sh.at[pl.ds(r0 + j * CHUNK, CHUNK)], zbuf_v)
        pltpu.sync_copy(
            zbuf_v, out_hbm.at[pl.ds(obase + AGG_ROWS + r0 + j * CHUNK, CHUNK)])


def _make_agg(feat):
    @functools.partial(
        pl.kernel,
        mesh=_mesh(),
        out_type=jax.ShapeDtypeStruct((NC, AGG_ROWS, feat), jnp.float32),
        scratch_types=[
            pltpu.VMEM((CHUNK,), jnp.int32),
            pltpu.VMEM((CHUNK,), jnp.int32),
            pltpu.VMEM((CHUNK, feat), jnp.float32),
            pltpu.VMEM_SHARED((AGG_ROWS, feat), jnp.float32),
            pltpu.SemaphoreType.DMA,
        ],
    )
    def agg_kernel(src_hbm, dst_hbm, h_hbm, out_hbm,
                   si_v, di_v, rows_v, agg_sh, sem):
        cid = lax.axis_index("c")
        sid = lax.axis_index("s")
        wid = sid * NC + cid
        r0 = sid * RPT

        def zbody(r, carry):
            for j in range(feat // 16):
                rows_v[r, pl.ds(j * 16, 16)] = jnp.zeros((16,), jnp.float32)
            return carry

        lax.fori_loop(0, CHUNK, zbody, 0)
        for j in range(RB):
            pltpu.sync_copy(rows_v, agg_sh.at[pl.ds(r0 + j * CHUNK, CHUNK)])
        plsc.subcore_barrier()

        def body(i, carry):
            off = wid * EPW + i * CHUNK
            pltpu.sync_copy(src_hbm.at[pl.ds(off, CHUNK)], si_v)
            pltpu.sync_copy(dst_hbm.at[pl.ds(off, CHUNK)], di_v)
            pltpu.async_copy(h_hbm.at[si_v], rows_v, sem).wait()
            pltpu.sync_copy(rows_v, agg_sh.at[di_v], add=True)
            return carry

        lax.fori_loop(0, N_CHUNKS, body, 0)
        plsc.subcore_barrier()
        for j in range(RB):
            pltpu.sync_copy(agg_sh.at[pl.ds(r0 + j * CHUNK, CHUNK)], rows_v)
            pltpu.sync_copy(rows_v, out_hbm.at[cid, pl.ds(r0 + j * CHUNK, CHUNK)])

    return agg_kernel


_agg128 = _make_agg(128)


def _mm0_body(f_ref, w_ref, d0_ref, d1_ref, o_ref):
    d = d0_ref[...] + d1_ref[...]
    s = lax.rsqrt(jnp.maximum(d, 1.0))[:N_NODES]
    o_ref[...] = jnp.dot(f_ref[...] * s, w_ref[...],
                         preferred_element_type=jnp.float32)


def _mmk_body(p_ref, dd0_ref, dd1_ref, b_ref, ds0_ref, ds1_ref, w_ref, o_ref):
    a = p_ref[0] + p_ref[1]
    sd = lax.rsqrt(jnp.maximum(dd0_ref[...] + dd1_ref[...], 1.0))
    h = jnp.maximum(a * sd + b_ref[...], 0.0)
    ss = lax.rsqrt(jnp.maximum(ds0_ref[...] + ds1_ref[...], 1.0))
    o_ref[...] = jnp.dot(h * ss, w_ref[...],
                         preferred_element_type=jnp.float32)


def _scale_body(p_ref, dd0_ref, dd1_ref, b_ref, ds0_ref, ds1_ref, o_ref):
    a = p_ref[0] + p_ref[1]
    sd = lax.rsqrt(jnp.maximum(dd0_ref[...] + dd1_ref[...], 1.0))
    h = jnp.maximum(a * sd + b_ref[...], 0.0)
    ss = lax.rsqrt(jnp.maximum(ds0_ref[...] + ds1_ref[...], 1.0))
    o_ref[...] = h * ss


def _fin_body(p_ref, dd0_ref, dd1_ref, b_ref, w_ref, o_ref):
    a = p_ref[0] + p_ref[1]
    sd = lax.rsqrt(jnp.maximum(dd0_ref[...] + dd1_ref[...], 1.0))
    z = (a * sd)[:N_NODES]
    o_ref[...] = jnp.dot(z, w_ref[...],
                         preferred_element_type=jnp.float32) + b_ref[...]


def kernel(features, edge_index, W0, b0, W1, b1, W2, b2):
    src = edge_index[0].astype(jnp.int32)
    dst = edge_index[1].astype(jnp.int32)
    pad = E_PAD - N_EDGES
    ar = jnp.arange(pad, dtype=jnp.int32)
    dummy = N_NODES + ar % (AGG_ROWS - N_NODES)
    src_deg = jnp.concatenate([src, dummy])
    dst_deg = jnp.concatenate([dst, dummy])
    src_agg = jnp.concatenate([src, (ar * 131) % N_NODES])
    dst_agg = dst_deg

    deg = _deg_kernel(src_deg, dst_deg).reshape(NC, 2, AGG_ROWS)
    ds0 = deg[0, 0].reshape(AGG_ROWS, 1)
    ds1 = deg[1, 0].reshape(AGG_ROWS, 1)
    dd0 = deg[0, 1].reshape(AGG_ROWS, 1)
    dd1 = deg[1, 1].reshape(AGG_ROWS, 1)

    h0 = pl.pallas_call(
        _mm0_body,
        out_shape=jax.ShapeDtypeStruct((N_NODES, 128), jnp.float32),
    )(features, W0, ds0, ds1)
    p0 = _agg128(src_agg, dst_agg, h0)

    h1 = pl.pallas_call(
        _mmk_body,
        out_shape=jax.ShapeDtypeStruct((AGG_ROWS, 128), jnp.float32),
    )(p0, dd0, dd1, b0.reshape(1, 128), ds0, ds1, W1)
    p1 = _agg128(src_agg, dst_agg, h1)

    h2 = pl.pallas_call(
        _scale_body,
        out_shape=jax.ShapeDtypeStruct((AGG_ROWS, 128), jnp.float32),
    )(p1, dd0, dd1, b1.reshape(1, 128), ds0, ds1)
    p2 = _agg128(src_agg, dst_agg, h2)

    out = pl.pallas_call(
        _fin_body,
        out_shape=jax.ShapeDtypeStruct((N_NODES, 64), jnp.float32),
    )(p2, dd0, dd1, b2.reshape(1, 64), W2)
    return out


# speedup vs baseline: 11.3036x; 2.1126x over previous
import functools

import jax
import jax.numpy as jnp
from jax import lax
from jax.experimental import pallas as pl
from jax.experimental.pallas import tpu as pltpu
from jax.experimental.pallas import tpu_sc as plsc

N_NODES = 10000
N_EDGES = 320000
NC = 2
NS = 16
NW = NC * NS
CHUNK = 128
N_CHUNKS = 80
EPW = N_CHUNKS * CHUNK
E_PAD = EPW * NW
AGG_ROWS = 10240
RPT = AGG_ROWS // NS
RB = RPT // CHUNK


def _mesh():
    return plsc.VectorSubcoreMesh(core_axis_name="c", subcore_axis_name="s")


@functools.partial(
    pl.kernel,
    mesh=_mesh(),
    out_type=jax.ShapeDtypeStruct((NC * 2 * AGG_ROWS,), jnp.float32),
    scratch_types=(
        [pltpu.VMEM((CHUNK,), jnp.int32) for _ in range(8)]
        + [pltpu.VMEM((CHUNK,), jnp.float32),
           pltpu.VMEM((CHUNK,), jnp.float32),
           pltpu.VMEM_SHARED((AGG_ROWS,), jnp.float32),
           pltpu.VMEM_SHARED((AGG_ROWS,), jnp.float32)]
        + [pltpu.SemaphoreType.DMA for _ in range(8)]
    ),
)
def _deg_kernel(src_hbm, dst_hbm, out_hbm,
                si0, si1, si2, si3, di0, di1, di2, di3,
                ones_v, zbuf_v, hs_sh, hd_sh,
                mi0, mi1, mi2, mi3, ma0, ma1, mb0, mb1):
    si = (si0, si1, si2, si3)
    di = (di0, di1, di2, di3)
    semi = (mi0, mi1, mi2, mi3)
    sema = (ma0, ma1)
    semb = (mb0, mb1)
    cid = lax.axis_index("c")
    sid = lax.axis_index("s")
    wid = sid * NC + cid
    r0 = sid * RPT
    base = wid * EPW

    def idx_start(c, q):
        off = base + c * CHUNK
        pltpu.async_copy(src_hbm.at[pl.ds(off, CHUNK)], si[q], semi[q])
        pltpu.async_copy(dst_hbm.at[pl.ds(off, CHUNK)], di[q], semi[q])

    def idx_start_wrap(c, q):
        off = base + lax.rem(c, N_CHUNKS) * CHUNK
        pltpu.async_copy(src_hbm.at[pl.ds(off, CHUNK)], si[q], semi[q])
        pltpu.async_copy(dst_hbm.at[pl.ds(off, CHUNK)], di[q], semi[q])

    def idx_wait(q):
        pltpu.make_async_copy(src_hbm.at[pl.ds(base, CHUNK)], si[q], semi[q]).wait()
        pltpu.make_async_copy(dst_hbm.at[pl.ds(base, CHUNK)], di[q], semi[q]).wait()

    def sa_start(q, p):
        pltpu.async_copy(ones_v, hs_sh.at[si[q]], sema[p], add=True)

    def sa_wait(q, p):
        pltpu.make_async_copy(ones_v, hs_sh.at[si[q]], sema[p]).wait()

    def sb_start(q, p):
        pltpu.async_copy(ones_v, hd_sh.at[di[q]], semb[p], add=True)

    def sb_wait(q, p):
        pltpu.make_async_copy(ones_v, hd_sh.at[di[q]], semb[p]).wait()

    for j in range(CHUNK // 16):
        ones_v[pl.ds(j * 16, 16)] = jnp.full((16,), 1.0, jnp.float32)
        zbuf_v[pl.ds(j * 16, 16)] = jnp.zeros((16,), jnp.float32)
    for j in range(RB):
        pltpu.sync_copy(zbuf_v, hs_sh.at[pl.ds(r0 + j * CHUNK, CHUNK)])
        pltpu.sync_copy(zbuf_v, hd_sh.at[pl.ds(r0 + j * CHUNK, CHUNK)])
    plsc.subcore_barrier()

    idx_start(0, 0)
    idx_start(1, 1)
    idx_wait(0); sa_start(0, 0); sb_start(0, 0); idx_start(2, 2)
    idx_wait(1); sa_start(1, 1); sb_start(1, 1); idx_start(3, 3)
    idx_wait(2); sa_wait(0, 0); sb_wait(0, 0)
    sa_start(2, 0); sb_start(2, 0); idx_start(4, 0)
    idx_wait(3); sa_wait(1, 1); sb_wait(1, 1)
    sa_start(3, 1); sb_start(3, 1); idx_start(5, 1)

    def body(g, carry):
        for b in range(4):
            c = 4 * g + b
            p = b % 2
            qp2 = (b + 2) % 4
            idx_wait(b)
            sa_wait(qp2, p)
            sb_wait(qp2, p)
            sa_start(b, p)
            sb_start(b, p)
            idx_start_wrap(c + 2, qp2)
        return carry

    lax.fori_loop(1, N_CHUNKS // 4, body, 0)
    sa_wait(2, 0); sb_wait(2, 0)
    sa_wait(3, 1); sb_wait(3, 1)
    idx_wait(0)
    idx_wait(1)
    plsc.subcore_barrier()
    obase = cid * (2 * AGG_ROWS)
    for j in range(RB):
        pltpu.sync_copy(hs_sh.at[pl.ds(r0 + j * CHUNK, CHUNK)], zbuf_v)
        pltpu.sync_copy(zbuf_v, out_hbm.at[pl.ds(obase + r0 + j * CHUNK, CHUNK)])
        pltpu.sync_copy(hd_sh.at[pl.ds(r0 + j * CHUNK, CHUNK)], zbuf_v)
        pltpu.sync_copy(
            zbuf_v, out_hbm.at[pl.ds(obase + AGG_ROWS + r0 + j * CHUNK, CHUNK)])


def _make_agg(feat):
    @functools.partial(
        pl.kernel,
        mesh=_mesh(),
        out_type=jax.ShapeDtypeStruct((NC, AGG_ROWS, feat), jnp.float32),
        scratch_types=(
            [pltpu.VMEM((CHUNK,), jnp.int32) for _ in range(8)]
            + [pltpu.VMEM((CHUNK, feat), jnp.float32),
               pltpu.VMEM((CHUNK, feat), jnp.float32),
               pltpu.VMEM_SHARED((AGG_ROWS, feat), jnp.float32)]
            + [pltpu.SemaphoreType.DMA for _ in range(8)]
        ),
    )
    def agg_kernel(src_hbm, dst_hbm, h_hbm, out_hbm,
                   si0, si1, si2, si3, di0, di1, di2, di3,
                   rows0, rows1, agg_sh,
                   mi0, mi1, mi2, mi3, mg0, mg1, ms0, ms1):
        si = (si0, si1, si2, si3)
        di = (di0, di1, di2, di3)
        rows = (rows0, rows1)
        semi = (mi0, mi1, mi2, mi3)
        semg = (mg0, mg1)
        sems = (ms0, ms1)
        cid = lax.axis_index("c")
        sid = lax.axis_index("s")
        wid = sid * NC + cid
        r0 = sid * RPT
        base = wid * EPW

        def idx_start(c, q):
            off = base + c * CHUNK
            pltpu.async_copy(src_hbm.at[pl.ds(off, CHUNK)], si[q], semi[q])
            pltpu.async_copy(dst_hbm.at[pl.ds(off, CHUNK)], di[q], semi[q])

        def idx_start_wrap(c, q):
            off = base + lax.rem(c, N_CHUNKS) * CHUNK
            pltpu.async_copy(src_hbm.at[pl.ds(off, CHUNK)], si[q], semi[q])
            pltpu.async_copy(dst_hbm.at[pl.ds(off, CHUNK)], di[q], semi[q])

        def idx_wait(q):
            pltpu.make_async_copy(
                src_hbm.at[pl.ds(base, CHUNK)], si[q], semi[q]).wait()
            pltpu.make_async_copy(
                dst_hbm.at[pl.ds(base, CHUNK)], di[q], semi[q]).wait()

        def gather_start(q, p):
            pltpu.async_copy(h_hbm.at[si[q]], rows[p], semg[p])

        def gather_wait(q, p):
            pltpu.make_async_copy(h_hbm.at[si[q]], rows[p], semg[p]).wait()

        def scat_start(q, p):
            pltpu.async_copy(rows[p], agg_sh.at[di[q]], sems[p], add=True)

        def scat_wait(q, p):
            pltpu.make_async_copy(rows[p], agg_sh.at[di[q]], sems[p]).wait()

        def zbody(r, carry):
            for j in range(feat // 16):
                rows0[r, pl.ds(j * 16, 16)] = jnp.zeros((16,), jnp.float32)
            return carry

        lax.fori_loop(0, CHUNK, zbody, 0)
        for j in range(RB):
            pltpu.sync_copy(rows0, agg_sh.at[pl.ds(r0 + j * CHUNK, CHUNK)])
        plsc.subcore_barrier()

        idx_start(0, 0)
        idx_start(1, 1)
        idx_wait(0); gather_start(0, 0); idx_start(2, 2)
        idx_wait(1); gather_start(1, 1)
        gather_wait(0, 0); scat_start(0, 0); idx_start(3, 3)
        idx_wait(2); scat_wait(0, 0); gather_start(2, 0)
        gather_wait(1, 1); scat_start(1, 1); idx_start(4, 0)
        idx_wait(3); scat_wait(1, 1); gather_start(3, 1)
        gather_wait(2, 0); scat_start(2, 0); idx_start(5, 1)

        def body(g, carry):
            for b in range(4):
                c = 4 * g + b
                p = b % 2
                qp2 = (b + 2) % 4
                qm1 = (b + 3) % 4
                idx_wait(b)
                scat_wait(qp2, p)
                gather_start(b, p)
                gather_wait(qm1, 1 - p)
                scat_start(qm1, 1 - p)
                idx_start_wrap(c + 2, qp2)
            return carry

        lax.fori_loop(1, N_CHUNKS // 4, body, 0)
        gather_wait(3, 1)
        scat_start(3, 1)
        scat_wait(2, 0)
        scat_wait(3, 1)
        idx_wait(0)
        idx_wait(1)
        plsc.subcore_barrier()
        for j in range(RB):
            pltpu.sync_copy(agg_sh.at[pl.ds(r0 + j * CHUNK, CHUNK)], rows0)
            pltpu.sync_copy(rows0, out_hbm.at[cid, pl.ds(r0 + j * CHUNK, CHUNK)])

    return agg_kernel


_agg128 = _make_agg(128)


def _mm0_body(f_ref, w_ref, d0_ref, d1_ref, o_ref):
    d = d0_ref[...] + d1_ref[...]
    s = lax.rsqrt(jnp.maximum(d, 1.0))[:N_NODES]
    o_ref[...] = jnp.dot(f_ref[...] * s, w_ref[...],
                         preferred_element_type=jnp.float32)


def _mmk_body(p_ref, dd0_ref, dd1_ref, b_ref, ds0_ref, ds1_ref, w_ref, o_ref):
    a = p_ref[0] + p_ref[1]
    sd = lax.rsqrt(jnp.maximum(dd0_ref[...] + dd1_ref[...], 1.0))
    h = jnp.maximum(a * sd + b_ref[...], 0.0)
    ss = lax.rsqrt(jnp.maximum(ds0_ref[...] + ds1_ref[...], 1.0))
    o_ref[...] = jnp.dot(h * ss, w_ref[...],
                         preferred_element_type=jnp.float32)


def _scale_body(p_ref, dd0_ref, dd1_ref, b_ref, ds0_ref, ds1_ref, o_ref):
    a = p_ref[0] + p_ref[1]
    sd = lax.rsqrt(jnp.maximum(dd0_ref[...] + dd1_ref[...], 1.0))
    h = jnp.maximum(a * sd + b_ref[...], 0.0)
    ss = lax.rsqrt(jnp.maximum(ds0_ref[...] + ds1_ref[...], 1.0))
    o_ref[...] = h * ss


def _fin_body(p_ref, dd0_ref, dd1_ref, b_ref, w_ref, o_ref):
    a = p_ref[0] + p_ref[1]
    sd = lax.rsqrt(jnp.maximum(dd0_ref[...] + dd1_ref[...], 1.0))
    z = (a * sd)[:N_NODES]
    o_ref[...] = jnp.dot(z, w_ref[...],
                         preferred_element_type=jnp.float32) + b_ref[...]


def kernel(features, edge_index, W0, b0, W1, b1, W2, b2):
    src = edge_index[0].astype(jnp.int32)
    dst = edge_index[1].astype(jnp.int32)
    pad = E_PAD - N_EDGES
    ar = jnp.arange(pad, dtype=jnp.int32)
    dummy = N_NODES + ar % (AGG_ROWS - N_NODES)
    src_deg = jnp.concatenate([src, dummy])
    dst_deg = jnp.concatenate([dst, dummy])
    src_agg = jnp.concatenate([src, (ar * 131) % N_NODES])
    dst_agg = dst_deg

    deg = _deg_kernel(src_deg, dst_deg).reshape(NC, 2, AGG_ROWS)
    ds0 = deg[0, 0].reshape(AGG_ROWS, 1)
    ds1 = deg[1, 0].reshape(AGG_ROWS, 1)
    dd0 = deg[0, 1].reshape(AGG_ROWS, 1)
    dd1 = deg[1, 1].reshape(AGG_ROWS, 1)

    h0 = pl.pallas_call(
        _mm0_body,
        out_shape=jax.ShapeDtypeStruct((N_NODES, 128), jnp.float32),
    )(features, W0, ds0, ds1)
    p0 = _agg128(src_agg, dst_agg, h0)

    h1 = pl.pallas_call(
        _mmk_body,
        out_shape=jax.ShapeDtypeStruct((AGG_ROWS, 128), jnp.float32),
    )(p0, dd0, dd1, b0.reshape(1, 128), ds0, ds1, W1)
    p1 = _agg128(src_agg, dst_agg, h1)

    h2 = pl.pallas_call(
        _scale_body,
        out_shape=jax.ShapeDtypeStruct((AGG_ROWS, 128), jnp.float32),
    )(p1, dd0, dd1, b1.reshape(1, 128), ds0, ds1)
    p2 = _agg128(src_agg, dst_agg, h2)

    out = pl.pallas_call(
        _fin_body,
        out_shape=jax.ShapeDtypeStruct((N_NODES, 64), jnp.float32),
    )(p2, dd0, dd1, b2.reshape(1, 64), W2)
    return out


# speedup vs baseline: 11.4176x; 1.0101x over previous
import functools

import jax
import jax.numpy as jnp
from jax import lax
from jax.experimental import pallas as pl
from jax.experimental.pallas import tpu as pltpu
from jax.experimental.pallas import tpu_sc as plsc

N_NODES = 10000
N_EDGES = 320000
NC = 2
NS = 16
NW = NC * NS
CHUNK = 128
N_CHUNKS = 80
EPW = N_CHUNKS * CHUNK
E_PAD = EPW * NW
AGG_ROWS = 10240
RPT = AGG_ROWS // NS
RB = RPT // CHUNK


def _mesh():
    return plsc.VectorSubcoreMesh(core_axis_name="c", subcore_axis_name="s")


@functools.partial(
    pl.kernel,
    mesh=_mesh(),
    out_type=jax.ShapeDtypeStruct((NC * 2 * AGG_ROWS,), jnp.float32),
    scratch_types=(
        [pltpu.VMEM((2, CHUNK), jnp.int32) for _ in range(4)]
        + [pltpu.VMEM((CHUNK,), jnp.float32),
           pltpu.VMEM((CHUNK,), jnp.float32),
           pltpu.VMEM_SHARED((AGG_ROWS,), jnp.float32),
           pltpu.VMEM_SHARED((AGG_ROWS,), jnp.float32)]
        + [pltpu.SemaphoreType.DMA for _ in range(8)]
    ),
)
def _deg_kernel(eidx_hbm, out_hbm,
                ib0, ib1, ib2, ib3, ones_v, zbuf_v, hs_sh, hd_sh,
                mi0, mi1, mi2, mi3, ma0, ma1, mb0, mb1):
    ib = (ib0, ib1, ib2, ib3)
    semi = (mi0, mi1, mi2, mi3)
    sema = (ma0, ma1)
    semb = (mb0, mb1)
    cid = lax.axis_index("c")
    sid = lax.axis_index("s")
    wid = sid * NC + cid
    r0 = sid * RPT
    cbase = wid * N_CHUNKS

    def idx_start(c, q):
        pltpu.async_copy(eidx_hbm.at[cbase + c], ib[q], semi[q])

    def idx_start_wrap(c, q):
        pltpu.async_copy(eidx_hbm.at[cbase + lax.rem(c, N_CHUNKS)], ib[q], semi[q])

    def idx_wait(q):
        pltpu.make_async_copy(eidx_hbm.at[cbase], ib[q], semi[q]).wait()

    def sa_start(q, p):
        pltpu.async_copy(ones_v, hs_sh.at[ib[q].at[0]], sema[p], add=True)

    def sa_wait(q, p):
        pltpu.make_async_copy(ones_v, hs_sh.at[ib[q].at[0]], sema[p]).wait()

    def sb_start(q, p):
        pltpu.async_copy(ones_v, hd_sh.at[ib[q].at[1]], semb[p], add=True)

    def sb_wait(q, p):
        pltpu.make_async_copy(ones_v, hd_sh.at[ib[q].at[1]], semb[p]).wait()

    for j in range(CHUNK // 16):
        ones_v[pl.ds(j * 16, 16)] = jnp.full((16,), 1.0, jnp.float32)
        zbuf_v[pl.ds(j * 16, 16)] = jnp.zeros((16,), jnp.float32)
    for j in range(RB):
        pltpu.sync_copy(zbuf_v, hs_sh.at[pl.ds(r0 + j * CHUNK, CHUNK)])
        pltpu.sync_copy(zbuf_v, hd_sh.at[pl.ds(r0 + j * CHUNK, CHUNK)])
    plsc.subcore_barrier()

    idx_start(0, 0)
    idx_start(1, 1)
    idx_wait(0); sa_start(0, 0); sb_start(0, 0); idx_start(2, 2)
    idx_wait(1); sa_start(1, 1); sb_start(1, 1); idx_start(3, 3)
    idx_wait(2); sa_wait(0, 0); sb_wait(0, 0)
    sa_start(2, 0); sb_start(2, 0); idx_start(4, 0)
    idx_wait(3); sa_wait(1, 1); sb_wait(1, 1)
    sa_start(3, 1); sb_start(3, 1); idx_start(5, 1)

    def body(g, carry):
        for b in range(4):
            c = 4 * g + b
            p = b % 2
            qp2 = (b + 2) % 4
            idx_wait(b)
            sa_wait(qp2, p)
            sb_wait(qp2, p)
            sa_start(b, p)
            sb_start(b, p)
            idx_start_wrap(c + 2, qp2)
        return carry

    lax.fori_loop(1, N_CHUNKS // 4, body, 0)
    sa_wait(2, 0); sb_wait(2, 0)
    sa_wait(3, 1); sb_wait(3, 1)
    idx_wait(0)
    idx_wait(1)
    plsc.subcore_barrier()
    obase = cid * (2 * AGG_ROWS)
    for j in range(RB):
        pltpu.sync_copy(hs_sh.at[pl.ds(r0 + j * CHUNK, CHUNK)], zbuf_v)
        pltpu.sync_copy(zbuf_v, out_hbm.at[pl.ds(obase + r0 + j * CHUNK, CHUNK)])
        pltpu.sync_copy(hd_sh.at[pl.ds(r0 + j * CHUNK, CHUNK)], zbuf_v)
        pltpu.sync_copy(
            zbuf_v, out_hbm.at[pl.ds(obase + AGG_ROWS + r0 + j * CHUNK, CHUNK)])


def _make_agg(feat):
    @functools.partial(
        pl.kernel,
        mesh=_mesh(),
        out_type=jax.ShapeDtypeStruct((NC, AGG_ROWS, feat), jnp.float32),
        scratch_types=(
            [pltpu.VMEM((2, CHUNK), jnp.int32) for _ in range(4)]
            + [pltpu.VMEM((CHUNK, feat), jnp.float32),
               pltpu.VMEM((CHUNK, feat), jnp.float32),
               pltpu.VMEM_SHARED((AGG_ROWS, feat), jnp.float32)]
            + [pltpu.SemaphoreType.DMA for _ in range(9)]
        ),
    )
    def agg_kernel(eidx_hbm, h_hbm, out_hbm,
                   ib0, ib1, ib2, ib3, rows0, rows1, agg_sh,
                   mi0, mi1, mi2, mi3, mg0, mg1, ms0, ms1, mo):
        ib = (ib0, ib1, ib2, ib3)
        rows = (rows0, rows1)
        semi = (mi0, mi1, mi2, mi3)
        semg = (mg0, mg1)
        sems = (ms0, ms1)
        cid = lax.axis_index("c")
        sid = lax.axis_index("s")
        wid = sid * NC + cid
        r0 = sid * RPT
        cbase = wid * N_CHUNKS

        def idx_start(c, q):
            pltpu.async_copy(eidx_hbm.at[cbase + c], ib[q], semi[q])

        def idx_start_wrap(c, q):
            pltpu.async_copy(
                eidx_hbm.at[cbase + lax.rem(c, N_CHUNKS)], ib[q], semi[q])

        def idx_wait(q):
            pltpu.make_async_copy(eidx_hbm.at[cbase], ib[q], semi[q]).wait()

        def gather_start(q, p):
            pltpu.async_copy(h_hbm.at[ib[q].at[0]], rows[p], semg[p])

        def gather_wait(q, p):
            pltpu.make_async_copy(h_hbm.at[ib[q].at[0]], rows[p], semg[p]).wait()

        def scat_start(q, p):
            pltpu.async_copy(rows[p], agg_sh.at[ib[q].at[1]], sems[p], add=True)

        def scat_wait(q, p):
            pltpu.make_async_copy(rows[p], agg_sh.at[ib[q].at[1]], sems[p]).wait()

        def zbody(r, carry):
            for j in range(feat // 16):
                rows0[r, pl.ds(j * 16, 16)] = jnp.zeros((16,), jnp.float32)
            return carry

        lax.fori_loop(0, CHUNK, zbody, 0)
        for j in range(RB):
            pltpu.async_copy(rows0, agg_sh.at[pl.ds(r0 + j * CHUNK, CHUNK)], mo)
        for j in range(RB):
            pltpu.make_async_copy(
                rows0, agg_sh.at[pl.ds(r0, CHUNK)], mo).wait()
        plsc.subcore_barrier()

        idx_start(0, 0)
        idx_start(1, 1)
        idx_wait(0); gather_start(0, 0); idx_start(2, 2)
        idx_wait(1); gather_start(1, 1)
        gather_wait(0, 0); scat_start(0, 0); idx_start(3, 3)
        idx_wait(2); scat_wait(0, 0); gather_start(2, 0)
        gather_wait(1, 1); scat_start(1, 1); idx_start(4, 0)
        idx_wait(3); scat_wait(1, 1); gather_start(3, 1)
        gather_wait(2, 0); scat_start(2, 0); idx_start(5, 1)

        def body(g, carry):
            for b in range(4):
                c = 4 * g + b
                p = b % 2
                qp2 = (b + 2) % 4
                qm1 = (b + 3) % 4
                idx_wait(b)
                scat_wait(qp2, p)
                gather_start(b, p)
                gather_wait(qm1, 1 - p)
                scat_start(qm1, 1 - p)
                idx_start_wrap(c + 2, qp2)
            return carry

        lax.fori_loop(1, N_CHUNKS // 4, body, 0)
        gather_wait(3, 1)
        scat_start(3, 1)
        scat_wait(2, 0)
        scat_wait(3, 1)
        idx_wait(0)
        idx_wait(1)
        plsc.subcore_barrier()
        for j in range(RB):
            p = j % 2
            if j >= 2:
                pltpu.make_async_copy(
                    rows[p], out_hbm.at[cid, pl.ds(r0, CHUNK)], sems[p]).wait()
            pltpu.sync_copy(agg_sh.at[pl.ds(r0 + j * CHUNK, CHUNK)], rows[p])
            pltpu.async_copy(
                rows[p], out_hbm.at[cid, pl.ds(r0 + j * CHUNK, CHUNK)], sems[p])
        for j in (RB - 2, RB - 1):
            p = j % 2
            pltpu.make_async_copy(
                rows[p], out_hbm.at[cid, pl.ds(r0, CHUNK)], sems[p]).wait()

    return agg_kernel


_agg128 = _make_agg(128)


def _mm0_body(f_ref, w_ref, d0_ref, d1_ref, o_ref):
    d = d0_ref[...] + d1_ref[...]
    s = lax.rsqrt(jnp.maximum(d, 1.0))[:N_NODES]
    o_ref[...] = jnp.dot(f_ref[...] * s, w_ref[...],
                         preferred_element_type=jnp.float32)


def _mmk_body(p_ref, dd0_ref, dd1_ref, b_ref, ds0_ref, ds1_ref, w_ref, o_ref):
    a = p_ref[0] + p_ref[1]
    sd = lax.rsqrt(jnp.maximum(dd0_ref[...] + dd1_ref[...], 1.0))
    h = jnp.maximum(a * sd + b_ref[...], 0.0)
    ss = lax.rsqrt(jnp.maximum(ds0_ref[...] + ds1_ref[...], 1.0))
    o_ref[...] = jnp.dot(h * ss, w_ref[...],
                         preferred_element_type=jnp.float32)


def _scale_body(p_ref, dd0_ref, dd1_ref, b_ref, ds0_ref, ds1_ref, o_ref):
    a = p_ref[0] + p_ref[1]
    sd = lax.rsqrt(jnp.maximum(dd0_ref[...] + dd1_ref[...], 1.0))
    h = jnp.maximum(a * sd + b_ref[...], 0.0)
    ss = lax.rsqrt(jnp.maximum(ds0_ref[...] + ds1_ref[...], 1.0))
    o_ref[...] = h * ss


def _fin_body(p_ref, dd0_ref, dd1_ref, b_ref, w_ref, o_ref):
    a = p_ref[0] + p_ref[1]
    sd = lax.rsqrt(jnp.maximum(dd0_ref[...] + dd1_ref[...], 1.0))
    z = (a * sd)[:N_NODES]
    o_ref[...] = jnp.dot(z, w_ref[...],
                         preferred_element_type=jnp.float32) + b_ref[...]


def kernel(features, edge_index, W0, b0, W1, b1, W2, b2):
    src = edge_index[0].astype(jnp.int32)
    dst = edge_index[1].astype(jnp.int32)
    pad = E_PAD - N_EDGES
    ar = jnp.arange(pad, dtype=jnp.int32)
    dummy = N_NODES + ar % (AGG_ROWS - N_NODES)
    src_deg = jnp.concatenate([src, dummy])
    dst_deg = jnp.concatenate([dst, dummy])
    src_agg = jnp.concatenate([src, (ar * 131) % N_NODES])
    dst_agg = dst_deg

    def packed(s, d):
        return jnp.stack(
            [s.reshape(NW * N_CHUNKS, CHUNK), d.reshape(NW * N_CHUNKS, CHUNK)],
            axis=1)

    eidx_deg = packed(src_deg, dst_deg)
    eidx_agg = packed(src_agg, dst_agg)

    deg = _deg_kernel(eidx_deg).reshape(NC, 2, AGG_ROWS)
    ds0 = deg[0, 0].reshape(AGG_ROWS, 1)
    ds1 = deg[1, 0].reshape(AGG_ROWS, 1)
    dd0 = deg[0, 1].reshape(AGG_ROWS, 1)
    dd1 = deg[1, 1].reshape(AGG_ROWS, 1)

    h0 = pl.pallas_call(
        _mm0_body,
        out_shape=jax.ShapeDtypeStruct((N_NODES, 128), jnp.float32),
    )(features, W0, ds0, ds1)
    p0 = _agg128(eidx_agg, h0)

    h1 = pl.pallas_call(
        _mmk_body,
        out_shape=jax.ShapeDtypeStruct((AGG_ROWS, 128), jnp.float32),
    )(p0, dd0, dd1, b0.reshape(1, 128), ds0, ds1, W1)
    p1 = _agg128(eidx_agg, h1)

    h2 = pl.pallas_call(
        _scale_body,
        out_shape=jax.ShapeDtypeStruct((AGG_ROWS, 128), jnp.float32),
    )(p1, dd0, dd1, b1.reshape(1, 128), ds0, ds1)
    p2 = _agg128(eidx_agg, h2)

    out = pl.pallas_call(
        _fin_body,
        out_shape=jax.ShapeDtypeStruct((N_NODES, 64), jnp.float32),
    )(p2, dd0, dd1, b2.reshape(1, 64), W2)
    return out
